# baseline (speedup 1.0000x reference)
import functools

import jax
import jax.numpy as jnp
from jax import lax
from jax.experimental import pallas as pl
from jax.experimental.pallas import tpu as pltpu
from jax.experimental.pallas import tpu_sc as plsc

D = 64
NC = 2
NS = 16
NW = NC * NS
LB = 128
SB = 8


def kernel(x, weight):
    B_, S_ = x.shape
    n_sb = S_ // SB
    n_jb = B_ // LB
    j_per_w = n_jb // NW
    n_pairs = S_ // 2

    x_l = x.T.reshape(n_sb, SB, n_jb, LB).transpose(0, 2, 1, 3)

    mesh = plsc.VectorSubcoreMesh(core_axis_name="c", subcore_axis_name="s")

    @functools.partial(
        pl.kernel,
        mesh=mesh,
        out_type=jax.ShapeDtypeStruct((S_, D // SB, n_jb, SB, LB), jnp.float32),
        compiler_params=pltpu.CompilerParams(
            use_tc_tiling_on_sc=False, needs_layout_passes=False
        ),
        scratch_types=[
            pltpu.VMEM((n_sb, SB, LB), jnp.int32),
            pltpu.VMEM((LB, D), jnp.float32),
            pltpu.VMEM((LB, D), jnp.float32),
            pltpu.VMEM((D // SB, SB, LB + 1), jnp.float32),
            pltpu.VMEM((D // SB, SB, LB + 1), jnp.float32),
            pltpu.SemaphoreType.DMA,
            pltpu.SemaphoreType.DMA,
            pltpu.SemaphoreType.DMA,
            pltpu.SemaphoreType.DMA,
            pltpu.SemaphoreType.DMA,
        ],
    )
    def k(x_hbm, table_hbm, out_hbm, idx_v, rows0, rows1, tr0, tr1,
          isem, gsem0, gsem1, osem0, osem1):
        rows = (rows0, rows1)
        trs = (tr0, tr1)
        gsems = (gsem0, gsem1)
        osems = (osem0, osem1)
        wid = lax.axis_index("s") * NC + lax.axis_index("c")
        lane_iota = lax.iota(jnp.int32, 16)

        def gather(s, p):
            return pltpu.async_copy(
                table_hbm.at[idx_v.at[s // SB, s % SB]], rows[p], gsems[p]
            )

        d_coords = [
            ((16 * k + lane_iota) // SB, (16 * k + lane_iota) % SB)
            for k in range(D // 16)
        ]

        def transpose(p):
            def b_body(b0, carry):
                bs = [b0 * 4 + bb for bb in range(4)]
                vs = []
                for b in bs:
                    for k in range(D // 16):
                        vs.append(rows[p][b, pl.ds(16 * k, 16)])
                i = 0
                for b in bs:
                    b_splat = jnp.zeros((16,), jnp.int32) + b
                    for k in range(D // 16):
                        plsc.store_scatter(
                            trs[p],
                            [d_coords[k][0], d_coords[k][1], b_splat],
                            vs[i],
                        )
                        i += 1
                return carry

            lax.fori_loop(0, LB // 4, b_body, 0)

        def run_j(j):
            pltpu.sync_copy(x_hbm.at[:, j], idx_v)

            def out_at(s):
                return out_hbm.at[s, :, j]

            gather(0, 0)

            def pair_body(g, carry):
                for p in range(2):
                    s = 2 * g + p
                    pltpu.make_async_copy(
                        table_hbm.at[idx_v.at[0, 0]], rows[p], gsems[p]
                    ).wait()
                    @pl.when(s + 1 < S_)
                    def _():
                        gather(s + 1, 1 - p)
                    @pl.when(g > 0)
                    def _():
                        pltpu.make_async_copy(
                            out_at(s), trs[p].at[:, :, pl.ds(0, LB)], osems[p]
                        ).wait()
                    transpose(p)
                    pltpu.async_copy(
                        trs[p].at[:, :, pl.ds(0, LB)], out_at(s), osems[p]
                    )
                return carry

            lax.fori_loop(0, n_pairs, pair_body, 0)

            for p in range(2):
                pltpu.make_async_copy(
                    out_at(p), trs[p].at[:, :, pl.ds(0, LB)], osems[p]
                ).wait()

        for jl in range(j_per_w):
            run_j(wid * j_per_w + jl)

    out_l = k(x_l, weight)
    return out_l.transpose(2, 4, 0, 1, 3).reshape(B_, S_, D)


# speedup vs baseline: 1.0039x; 1.0039x over previous
import functools

import jax
import jax.numpy as jnp
from jax import lax
from jax.experimental import pallas as pl
from jax.experimental.pallas import tpu as pltpu
from jax.experimental.pallas import tpu_sc as plsc

D = 64
NC = 2
NS = 16
NW = NC * NS
LB = 128
SB = 8


def kernel(x, weight):
    B_, S_ = x.shape
    n_sb = S_ // SB
    n_jb = B_ // LB
    j_per_w = n_jb // NW
    n_pairs = S_ // 2

    x_l = x.T.reshape(n_sb, SB, n_jb, LB).transpose(0, 2, 1, 3)

    mesh = plsc.VectorSubcoreMesh(core_axis_name="c", subcore_axis_name="s")

    @functools.partial(
        pl.kernel,
        mesh=mesh,
        out_type=jax.ShapeDtypeStruct((S_, D // SB, n_jb, SB, LB), jnp.float32),
        compiler_params=pltpu.CompilerParams(
            use_tc_tiling_on_sc=False, needs_layout_passes=False
        ),
        scratch_types=[
            pltpu.VMEM((n_sb, SB, LB), jnp.int32),
            pltpu.VMEM((LB, D), jnp.float32),
            pltpu.VMEM((LB, D), jnp.float32),
            pltpu.VMEM((LB, D), jnp.float32),
            pltpu.VMEM((LB, D), jnp.float32),
            pltpu.VMEM((D // SB, SB, LB + 1), jnp.float32),
            pltpu.VMEM((D // SB, SB, LB + 1), jnp.float32),
            pltpu.SemaphoreType.DMA,
            pltpu.SemaphoreType.DMA,
            pltpu.SemaphoreType.DMA,
            pltpu.SemaphoreType.DMA,
            pltpu.SemaphoreType.DMA,
            pltpu.SemaphoreType.DMA,
            pltpu.SemaphoreType.DMA,
        ],
    )
    def k(x_hbm, table_hbm, out_hbm, idx_v, rows0, rows1, rows2, rows3,
          tr0, tr1, isem, gsem0, gsem1, gsem2, gsem3, osem0, osem1):
        rows = (rows0, rows1, rows2, rows3)
        trs = (tr0, tr1)
        gsems = (gsem0, gsem1, gsem2, gsem3)
        osems = (osem0, osem1)
        wid = lax.axis_index("s") * NC + lax.axis_index("c")
        lane_iota = lax.iota(jnp.int32, 16)

        def gather(s, p):
            return pltpu.async_copy(
                table_hbm.at[idx_v.at[s // SB, s % SB]], rows[p], gsems[p]
            )

        d_coords = [
            ((16 * k + lane_iota) // SB, (16 * k + lane_iota) % SB)
            for k in range(D // 16)
        ]

        def transpose_from(p, t):
            def b_body(b0, carry):
                bs = [b0 * 4 + bb for bb in range(4)]
                vs = []
                for b in bs:
                    for k in range(D // 16):
                        vs.append(rows[p][b, pl.ds(16 * k, 16)])
                i = 0
                for b in bs:
                    b_splat = jnp.zeros((16,), jnp.int32) + b
                    for k in range(D // 16):
                        plsc.store_scatter(
                            trs[t],
                            [d_coords[k][0], d_coords[k][1], b_splat],
                            vs[i],
                        )
                        i += 1
                return carry

            lax.fori_loop(0, LB // 4, b_body, 0)

        def run_j(j):
            pltpu.sync_copy(x_hbm.at[:, j], idx_v)

            def out_at(s):
                return out_hbm.at[s, :, j]

            for p in range(4):
                gather(p, p)

            def quad_body(g, carry):
                for p in range(4):
                    s = 4 * g + p
                    t = p % 2
                    pltpu.make_async_copy(
                        table_hbm.at[idx_v.at[0, 0]], rows[p], gsems[p]
                    ).wait()
                    @pl.when(s > 1)
                    def _():
                        pltpu.make_async_copy(
                            out_at(s), trs[t].at[:, :, pl.ds(0, LB)], osems[t]
                        ).wait()
                    transpose_from(p, t)
                    @pl.when(s + 4 < S_)
                    def _():
                        gather(s + 4, p)
                    pltpu.async_copy(
                        trs[t].at[:, :, pl.ds(0, LB)], out_at(s), osems[t]
                    )
                return carry

            lax.fori_loop(0, S_ // 4, quad_body, 0)

            for p in range(2):
                pltpu.make_async_copy(
                    out_at(p), trs[p].at[:, :, pl.ds(0, LB)], osems[p]
                ).wait()

        for jl in range(j_per_w):
            run_j(wid * j_per_w + jl)

    out_l = k(x_l, weight)
    return out_l.transpose(2, 4, 0, 1, 3).reshape(B_, S_, D)


# speedup vs baseline: 1.0078x; 1.0039x over previous
import functools

import jax
import jax.numpy as jnp
from jax import lax
from jax.experimental import pallas as pl
from jax.experimental.pallas import tpu as pltpu
from jax.experimental.pallas import tpu_sc as plsc

D = 64
NC = 2
NS = 16
NW = NC * NS
LB = 128
SB = 8


def kernel(x, weight):
    B_, S_ = x.shape
    n_sb = S_ // SB
    n_jb = B_ // LB
    j_per_w = n_jb // NW
    n_pairs = S_ // 2

    x_l = x.T.reshape(n_sb, SB, n_jb, LB).transpose(0, 2, 1, 3)

    mesh = plsc.VectorSubcoreMesh(core_axis_name="c", subcore_axis_name="s")

    @functools.partial(
        pl.kernel,
        mesh=mesh,
        out_type=jax.ShapeDtypeStruct((S_, D // SB, n_jb, SB, LB), jnp.float32),
        compiler_params=pltpu.CompilerParams(
            use_tc_tiling_on_sc=False, needs_layout_passes=False
        ),
        scratch_types=[
            pltpu.VMEM((n_sb, SB, LB), jnp.int32),
            pltpu.VMEM((LB, D), jnp.float32),
            pltpu.VMEM((LB, D), jnp.float32),
            pltpu.VMEM((LB, D), jnp.float32),
            pltpu.VMEM((LB, D), jnp.float32),
            pltpu.VMEM((D // SB, SB, LB + 1), jnp.float32),
            pltpu.VMEM((D // SB, SB, LB + 1), jnp.float32),
            pltpu.SemaphoreType.DMA,
            pltpu.SemaphoreType.DMA,
            pltpu.SemaphoreType.DMA,
            pltpu.SemaphoreType.DMA,
            pltpu.SemaphoreType.DMA,
            pltpu.SemaphoreType.DMA,
            pltpu.SemaphoreType.DMA,
        ],
    )
    def k(x_hbm, table_hbm, out_hbm, idx_v, rows0, rows1, rows2, rows3,
          tr0, tr1, isem, gsem0, gsem1, gsem2, gsem3, osem0, osem1):
        rows = (rows0, rows1, rows2, rows3)
        trs = (tr0, tr1)
        gsems = (gsem0, gsem1, gsem2, gsem3)
        osems = (osem0, osem1)
        wid = lax.axis_index("s") * NC + lax.axis_index("c")
        lane_iota = lax.iota(jnp.int32, 16)

        def gather(s, p):
            return pltpu.async_copy(
                table_hbm.at[idx_v.at[s // SB, s % SB]], rows[p], gsems[p]
            )

        d_coords = [
            ((16 * k + lane_iota) // SB, (16 * k + lane_iota) % SB)
            for k in range(D // 16)
        ]

        def transpose_from(p, t):
            def b_body(b0, carry):
                for half in range(2):
                    bs = [b0 * 8 + half * 4 + bb for bb in range(4)]
                    vs = []
                    for b in bs:
                        for k in range(D // 16):
                            vs.append(rows[p][b, pl.ds(16 * k, 16)])
                    i = 0
                    for b in bs:
                        b_splat = jnp.zeros((16,), jnp.int32) + b
                        for k in range(D // 16):
                            plsc.store_scatter(
                                trs[t],
                                [d_coords[k][0], d_coords[k][1], b_splat],
                                vs[i],
                            )
                            i += 1
                return carry

            lax.fori_loop(0, LB // 8, b_body, 0)

        def run_j(j):
            pltpu.sync_copy(x_hbm.at[:, j], idx_v)

            def out_at(s):
                return out_hbm.at[s, :, j]

            for p in range(4):
                gather(p, p)

            def quad_body(g, carry):
                for p in range(4):
                    s = 4 * g + p
                    t = p % 2
                    pltpu.make_async_copy(
                        table_hbm.at[idx_v.at[0, 0]], rows[p], gsems[p]
                    ).wait()
                    @pl.when(s > 1)
                    def _():
                        pltpu.make_async_copy(
                            out_at(s), trs[t].at[:, :, pl.ds(0, LB)], osems[t]
                        ).wait()
                    transpose_from(p, t)
                    @pl.when(s + 4 < S_)
                    def _():
                        gather(s + 4, p)
                    pltpu.async_copy(
                        trs[t].at[:, :, pl.ds(0, LB)], out_at(s), osems[t]
                    )
                return carry

            lax.fori_loop(0, S_ // 4, quad_body, 0)

            for p in range(2):
                pltpu.make_async_copy(
                    out_at(p), trs[p].at[:, :, pl.ds(0, LB)], osems[p]
                ).wait()

        for jl in range(j_per_w):
            run_j(wid * j_per_w + jl)

    out_l = k(x_l, weight)
    return out_l.transpose(2, 4, 0, 1, 3).reshape(B_, S_, D)


# speedup vs baseline: 1.1002x; 1.0917x over previous
import functools
import math

import jax
import jax.numpy as jnp
from jax import lax
from jax.experimental import pallas as pl
from jax.experimental.pallas import tpu as pltpu
from jax.experimental.pallas import tpu_sc as plsc

D = 64
NC = 2
NS = 16
NW = NC * NS
LB = 128
SB = 8


H_PACK = 500736
NUM_ROWS_STATIC = 1000000
RB = 1024


def _wpack(wt):
    G = H_PACK // RB

    def body(x1_ref, x2_ref, o_ref):
        o_ref[...] = jnp.concatenate([x1_ref[...].T, x2_ref[...].T], axis=1)

    last_blk = (NUM_ROWS_STATIC - 1) // RB

    return pl.pallas_call(
        body, grid=(G,),
        in_specs=[pl.BlockSpec((D, RB), lambda i: (0, i)),
                  pl.BlockSpec((D, RB),
                               lambda i: (0, jnp.minimum(i + G, last_blk)))],
        out_specs=pl.BlockSpec((RB, 2 * D), lambda i: (i, 0)),
        out_shape=jax.ShapeDtypeStruct((H_PACK, 2 * D), jnp.float32),
    )(wt, wt)


def _xmap(xt):
    def body(x_ref, o_ref):
        v = x_ref[...]
        ge = (v >= H_PACK).astype(jnp.int32)
        o_ref[...] = 2 * v - (2 * H_PACK - 1) * ge

    S_, B_ = xt.shape
    return pl.pallas_call(
        body, grid=(8,),
        in_specs=[pl.BlockSpec((S_, B_ // 8), lambda i: (0, i))],
        out_specs=pl.BlockSpec((S_, B_ // 8), lambda i: (0, i)),
        out_shape=jax.ShapeDtypeStruct((S_, B_), jnp.int32),
    )(xt)


def kernel(x, weight):
    B_, S_ = x.shape
    n_sb = S_ // SB
    n_jb = B_ // LB
    j_per_w = n_jb // NW
    n_pairs = S_ // 2

    x_u = _xmap(x.T)
    x_l = x_u.reshape(n_sb, SB, n_jb, LB).transpose(0, 2, 1, 3)
    table_lin = _wpack(weight.T).reshape(2 * H_PACK, D)

    mesh = plsc.VectorSubcoreMesh(core_axis_name="c", subcore_axis_name="s")

    @functools.partial(
        pl.kernel,
        mesh=mesh,
        out_type=jax.ShapeDtypeStruct((S_, D // SB, n_jb, SB, LB), jnp.float32),
        compiler_params=pltpu.CompilerParams(
            use_tc_tiling_on_sc=False, needs_layout_passes=False
        ),
        scratch_types=[
            pltpu.VMEM((n_sb, SB, LB), jnp.int32),
            pltpu.VMEM((LB, D), jnp.float32),
            pltpu.VMEM((LB, D), jnp.float32),
            pltpu.VMEM((LB, D), jnp.float32),
            pltpu.VMEM((LB, D), jnp.float32),
            pltpu.VMEM((D // SB, SB, LB + 1), jnp.float32),
            pltpu.VMEM((D // SB, SB, LB + 1), jnp.float32),
            pltpu.SemaphoreType.DMA,
            pltpu.SemaphoreType.DMA,
            pltpu.SemaphoreType.DMA,
            pltpu.SemaphoreType.DMA,
            pltpu.SemaphoreType.DMA,
            pltpu.SemaphoreType.DMA,
            pltpu.SemaphoreType.DMA,
        ],
    )
    def k(x_hbm, table_hbm, out_hbm, idx_v, rows0, rows1, rows2, rows3,
          tr0, tr1, isem, gsem0, gsem1, gsem2, gsem3, osem0, osem1):
        rows = (rows0, rows1, rows2, rows3)
        trs = (tr0, tr1)
        gsems = (gsem0, gsem1, gsem2, gsem3)
        osems = (osem0, osem1)
        wid = lax.axis_index("s") * NC + lax.axis_index("c")
        lane_iota = lax.iota(jnp.int32, 16)

        def gather(s, p):
            return pltpu.async_copy(
                table_hbm.at[idx_v.at[s // SB, s % SB]], rows[p], gsems[p]
            )

        d_coords = [
            ((16 * k + lane_iota) // SB, (16 * k + lane_iota) % SB)
            for k in range(D // 16)
        ]

        def transpose_from(p, t):
            def b_body(b0, carry):
                for half in range(2):
                    bs = [b0 * 8 + half * 4 + bb for bb in range(4)]
                    vs = []
                    for b in bs:
                        for k in range(D // 16):
                            vs.append(rows[p][b, pl.ds(16 * k, 16)])
                    i = 0
                    for b in bs:
                        b_splat = jnp.zeros((16,), jnp.int32) + b
                        for k in range(D // 16):
                            plsc.store_scatter(
                                trs[t],
                                [d_coords[k][0], d_coords[k][1], b_splat],
                                vs[i],
                            )
                            i += 1
                return carry

            lax.fori_loop(0, LB // 8, b_body, 0)

        def run_j(j):
            pltpu.sync_copy(x_hbm.at[:, j], idx_v)

            def out_at(s):
                return out_hbm.at[s, :, j]

            for p in range(4):
                gather(p, p)

            def quad_body(g, carry):
                for p in range(4):
                    s = 4 * g + p
                    t = p % 2
                    pltpu.make_async_copy(
                        table_hbm.at[idx_v.at[0, 0]], rows[p], gsems[p]
                    ).wait()
                    @pl.when(s > 1)
                    def _():
                        pltpu.make_async_copy(
                            out_at(s), trs[t].at[:, :, pl.ds(0, LB)], osems[t]
                        ).wait()
                    transpose_from(p, t)
                    @pl.when(s + 4 < S_)
                    def _():
                        gather(s + 4, p)
                    pltpu.async_copy(
                        trs[t].at[:, :, pl.ds(0, LB)], out_at(s), osems[t]
                    )
                return carry

            lax.fori_loop(0, S_ // 4, quad_body, 0)

            for p in range(2):
                pltpu.make_async_copy(
                    out_at(p), trs[p].at[:, :, pl.ds(0, LB)], osems[p]
                ).wait()

        for jl in range(j_per_w):
            run_j(wid * j_per_w + jl)

    out_l = k(x_l, table_lin)
    return out_l.transpose(2, 4, 0, 1, 3).reshape(B_, S_, D)


# speedup vs baseline: 1.1922x; 1.0836x over previous
import functools
import math

import jax
import jax.numpy as jnp
from jax import lax
from jax.experimental import pallas as pl
from jax.experimental.pallas import tpu as pltpu
from jax.experimental.pallas import tpu_sc as plsc

D = 64
NC = 2
NS = 16
NW = NC * NS
LB = 128
SB = 8


H_PACK = 501760
NUM_ROWS_STATIC = 1000000
RB = 2048


def _wpack(wt):
    G = H_PACK // RB

    def body(x1_ref, x2_ref, o_ref):
        o_ref[...] = jnp.concatenate([x1_ref[...].T, x2_ref[...].T], axis=1)

    last_blk = (NUM_ROWS_STATIC - 1) // RB

    return pl.pallas_call(
        body, grid=(G,),
        in_specs=[pl.BlockSpec((D, RB), lambda i: (0, i)),
                  pl.BlockSpec((D, RB),
                               lambda i: (0, jnp.minimum(i + G, last_blk)))],
        out_specs=pl.BlockSpec((RB, 2 * D), lambda i: (i, 0)),
        out_shape=jax.ShapeDtypeStruct((H_PACK, 2 * D), jnp.float32),
    )(wt, wt)


def _xmap(xt):
    def body(x_ref, o_ref):
        v = x_ref[...]
        ge = (v >= H_PACK).astype(jnp.int32)
        o_ref[...] = 2 * v - (2 * H_PACK - 1) * ge

    S_, B_ = xt.shape
    return pl.pallas_call(
        body, grid=(8,),
        in_specs=[pl.BlockSpec((S_, B_ // 8), lambda i: (0, i))],
        out_specs=pl.BlockSpec((S_, B_ // 8), lambda i: (0, i)),
        out_shape=jax.ShapeDtypeStruct((S_, B_), jnp.int32),
    )(xt)


def kernel(x, weight):
    B_, S_ = x.shape
    n_sb = S_ // SB
    n_jb = B_ // LB
    j_per_w = n_jb // NW
    n_pairs = S_ // 2

    x_u = _xmap(x.T)
    x_l = x_u.reshape(n_sb, SB, n_jb, LB).transpose(0, 2, 1, 3)
    table_lin = _wpack(weight.T).reshape(2 * H_PACK, D)

    mesh = plsc.VectorSubcoreMesh(core_axis_name="c", subcore_axis_name="s")

    @functools.partial(
        pl.kernel,
        mesh=mesh,
        out_type=jax.ShapeDtypeStruct((S_, D // SB, n_jb, SB, LB), jnp.float32),
        compiler_params=pltpu.CompilerParams(
            use_tc_tiling_on_sc=False, needs_layout_passes=False
        ),
        scratch_types=[
            pltpu.VMEM((n_sb, SB, LB), jnp.int32),
            pltpu.VMEM((LB, D), jnp.float32),
            pltpu.VMEM((LB, D), jnp.float32),
            pltpu.VMEM((LB, D), jnp.float32),
            pltpu.VMEM((LB, D), jnp.float32),
            pltpu.VMEM((D // SB, SB, LB + 1), jnp.float32),
            pltpu.VMEM((D // SB, SB, LB + 1), jnp.float32),
            pltpu.SemaphoreType.DMA,
            pltpu.SemaphoreType.DMA,
            pltpu.SemaphoreType.DMA,
            pltpu.SemaphoreType.DMA,
            pltpu.SemaphoreType.DMA,
            pltpu.SemaphoreType.DMA,
            pltpu.SemaphoreType.DMA,
        ],
    )
    def k(x_hbm, table_hbm, out_hbm, idx_v, rows0, rows1, rows2, rows3,
          tr0, tr1, isem, gsem0, gsem1, gsem2, gsem3, osem0, osem1):
        rows = (rows0, rows1, rows2, rows3)
        trs = (tr0, tr1)
        gsems = (gsem0, gsem1, gsem2, gsem3)
        osems = (osem0, osem1)
        wid = lax.axis_index("s") * NC + lax.axis_index("c")
        lane_iota = lax.iota(jnp.int32, 16)

        def gather(s, p):
            return pltpu.async_copy(
                table_hbm.at[idx_v.at[s // SB, s % SB]], rows[p], gsems[p]
            )

        d_coords = [
            ((16 * k + lane_iota) // SB, (16 * k + lane_iota) % SB)
            for k in range(D // 16)
        ]

        def transpose_from(p, t):
            def b_body(b0, carry):
                for half in range(2):
                    bs = [b0 * 8 + half * 4 + bb for bb in range(4)]
                    vs = []
                    for b in bs:
                        for k in range(D // 16):
                            vs.append(rows[p][b, pl.ds(16 * k, 16)])
                    i = 0
                    for b in bs:
                        b_splat = jnp.zeros((16,), jnp.int32) + b
                        for k in range(D // 16):
                            plsc.store_scatter(
                                trs[t],
                                [d_coords[k][0], d_coords[k][1], b_splat],
                                vs[i],
                            )
                            i += 1
                return carry

            lax.fori_loop(0, LB // 8, b_body, 0)

        def run_j(j):
            pltpu.sync_copy(x_hbm.at[:, j], idx_v)

            def out_at(s):
                return out_hbm.at[s, :, j]

            for p in range(4):
                gather(p, p)

            def quad_body(g, carry):
                for p in range(4):
                    s = 4 * g + p
                    t = p % 2
                    pltpu.make_async_copy(
                        table_hbm.at[idx_v.at[0, 0]], rows[p], gsems[p]
                    ).wait()
                    @pl.when(s > 1)
                    def _():
                        pltpu.make_async_copy(
                            out_at(s), trs[t].at[:, :, pl.ds(0, LB)], osems[t]
                        ).wait()
                    transpose_from(p, t)
                    @pl.when(s + 4 < S_)
                    def _():
                        gather(s + 4, p)
                    pltpu.async_copy(
                        trs[t].at[:, :, pl.ds(0, LB)], out_at(s), osems[t]
                    )
                return carry

            lax.fori_loop(0, S_ // 4, quad_body, 0)

            for p in range(2):
                pltpu.make_async_copy(
                    out_at(p), trs[p].at[:, :, pl.ds(0, LB)], osems[p]
                ).wait()

        for jl in range(j_per_w):
            run_j(wid * j_per_w + jl)

    out_l = k(x_l, table_lin)
    return out_l.transpose(2, 4, 0, 1, 3).reshape(B_, S_, D)


# speedup vs baseline: 1.2521x; 1.0502x over previous
import functools
import math

import jax
import jax.numpy as jnp
from jax import lax
from jax.experimental import pallas as pl
from jax.experimental.pallas import tpu as pltpu
from jax.experimental.pallas import tpu_sc as plsc

D = 64
NC = 2
NS = 16
NW = NC * NS
LB = 128
SB = 8


H_PACK = 503808
NUM_ROWS_STATIC = 1000000
RB = 4096


def _wpack(wt):
    G = H_PACK // RB

    def body(x1_ref, x2_ref, o_ref):
        o_ref[...] = jnp.concatenate([x1_ref[...].T, x2_ref[...].T], axis=1)

    last_blk = (NUM_ROWS_STATIC - 1) // RB

    return pl.pallas_call(
        body, grid=(G,),
        in_specs=[pl.BlockSpec((D, RB), lambda i: (0, i)),
                  pl.BlockSpec((D, RB),
                               lambda i: (0, jnp.minimum(i + G, last_blk)))],
        out_specs=pl.BlockSpec((RB, 2 * D), lambda i: (i, 0)),
        out_shape=jax.ShapeDtypeStruct((H_PACK, 2 * D), jnp.float32),
    )(wt, wt)


def _xmap(xt):
    def body(x_ref, o_ref):
        v = x_ref[...]
        ge = (v >= H_PACK).astype(jnp.int32)
        o_ref[...] = 2 * v - (2 * H_PACK - 1) * ge

    S_, B_ = xt.shape
    return pl.pallas_call(
        body, grid=(8,),
        in_specs=[pl.BlockSpec((S_, B_ // 8), lambda i: (0, i))],
        out_specs=pl.BlockSpec((S_, B_ // 8), lambda i: (0, i)),
        out_shape=jax.ShapeDtypeStruct((S_, B_), jnp.int32),
    )(xt)


def kernel(x, weight):
    B_, S_ = x.shape
    n_sb = S_ // SB
    n_jb = B_ // LB
    j_per_w = n_jb // NW
    n_pairs = S_ // 2

    x_u = _xmap(x.T)
    x_l = x_u.reshape(n_sb, SB, n_jb, LB).transpose(0, 2, 1, 3)
    table_lin = _wpack(weight.T).reshape(2 * H_PACK, D)

    mesh = plsc.VectorSubcoreMesh(core_axis_name="c", subcore_axis_name="s")

    @functools.partial(
        pl.kernel,
        mesh=mesh,
        out_type=jax.ShapeDtypeStruct((S_, D // SB, n_jb, SB, LB), jnp.float32),
        compiler_params=pltpu.CompilerParams(
            use_tc_tiling_on_sc=False, needs_layout_passes=False
        ),
        scratch_types=[
            pltpu.VMEM((n_sb, SB, LB), jnp.int32),
            pltpu.VMEM((LB, D), jnp.float32),
            pltpu.VMEM((LB, D), jnp.float32),
            pltpu.VMEM((LB, D), jnp.float32),
            pltpu.VMEM((LB, D), jnp.float32),
            pltpu.VMEM((D // SB, SB, LB + 1), jnp.float32),
            pltpu.VMEM((D // SB, SB, LB + 1), jnp.float32),
            pltpu.SemaphoreType.DMA,
            pltpu.SemaphoreType.DMA,
            pltpu.SemaphoreType.DMA,
            pltpu.SemaphoreType.DMA,
            pltpu.SemaphoreType.DMA,
            pltpu.SemaphoreType.DMA,
            pltpu.SemaphoreType.DMA,
        ],
    )
    def k(x_hbm, table_hbm, out_hbm, idx_v, rows0, rows1, rows2, rows3,
          tr0, tr1, isem, gsem0, gsem1, gsem2, gsem3, osem0, osem1):
        rows = (rows0, rows1, rows2, rows3)
        trs = (tr0, tr1)
        gsems = (gsem0, gsem1, gsem2, gsem3)
        osems = (osem0, osem1)
        wid = lax.axis_index("s") * NC + lax.axis_index("c")
        lane_iota = lax.iota(jnp.int32, 16)

        def gather(s, p):
            return pltpu.async_copy(
                table_hbm.at[idx_v.at[s // SB, s % SB]], rows[p], gsems[p]
            )

        d_coords = [
            ((16 * k + lane_iota) // SB, (16 * k + lane_iota) % SB)
            for k in range(D // 16)
        ]

        def transpose_from(p, t):
            def b_body(b0, carry):
                for half in range(2):
                    bs = [b0 * 8 + half * 4 + bb for bb in range(4)]
                    vs = []
                    for b in bs:
                        for k in range(D // 16):
                            vs.append(rows[p][b, pl.ds(16 * k, 16)])
                    i = 0
                    for b in bs:
                        b_splat = jnp.zeros((16,), jnp.int32) + b
                        for k in range(D // 16):
                            plsc.store_scatter(
                                trs[t],
                                [d_coords[k][0], d_coords[k][1], b_splat],
                                vs[i],
                            )
                            i += 1
                return carry

            lax.fori_loop(0, LB // 8, b_body, 0)

        def run_j(j):
            pltpu.sync_copy(x_hbm.at[:, j], idx_v)

            def out_at(s):
                return out_hbm.at[s, :, j]

            for p in range(4):
                gather(p, p)

            def quad_body(g, carry):
                for p in range(4):
                    s = 4 * g + p
                    t = p % 2
                    pltpu.make_async_copy(
                        table_hbm.at[idx_v.at[0, 0]], rows[p], gsems[p]
                    ).wait()
                    @pl.when(s > 1)
                    def _():
                        pltpu.make_async_copy(
                            out_at(s), trs[t].at[:, :, pl.ds(0, LB)], osems[t]
                        ).wait()
                    transpose_from(p, t)
                    @pl.when(s + 4 < S_)
                    def _():
                        gather(s + 4, p)
                    pltpu.async_copy(
                        trs[t].at[:, :, pl.ds(0, LB)], out_at(s), osems[t]
                    )
                return carry

            lax.fori_loop(0, S_ // 4, quad_body, 0)

            for p in range(2):
                pltpu.make_async_copy(
                    out_at(p), trs[p].at[:, :, pl.ds(0, LB)], osems[p]
                ).wait()

        for jl in range(j_per_w):
            run_j(wid * j_per_w + jl)

    out_l = k(x_l, table_lin)
    return out_l.transpose(2, 4, 0, 1, 3).reshape(B_, S_, D)


# speedup vs baseline: 1.2872x; 1.0281x over previous
import functools
import math

import jax
import jax.numpy as jnp
from jax import lax
from jax.experimental import pallas as pl
from jax.experimental.pallas import tpu as pltpu
from jax.experimental.pallas import tpu_sc as plsc

D = 64
NC = 2
NS = 16
NW = NC * NS
LB = 128
SB = 8


H_PACK = 507904
NUM_ROWS_STATIC = 1000000
RB = 8192


def _wpack(wt):
    G = H_PACK // RB

    def body(x1_ref, x2_ref, o_ref):
        o_ref[...] = jnp.concatenate([x1_ref[...].T, x2_ref[...].T], axis=1)

    last_blk = (NUM_ROWS_STATIC - 1) // RB

    return pl.pallas_call(
        body, grid=(G,),
        in_specs=[pl.BlockSpec((D, RB), lambda i: (0, i)),
                  pl.BlockSpec((D, RB),
                               lambda i: (0, jnp.minimum(i + G, last_blk)))],
        out_specs=pl.BlockSpec((RB, 2 * D), lambda i: (i, 0)),
        out_shape=jax.ShapeDtypeStruct((H_PACK, 2 * D), jnp.float32),
    )(wt, wt)


def _xmap(xt):
    def body(x_ref, o_ref):
        v = x_ref[...]
        ge = (v >= H_PACK).astype(jnp.int32)
        o_ref[...] = 2 * v - (2 * H_PACK - 1) * ge

    S_, B_ = xt.shape
    return pl.pallas_call(
        body, grid=(8,),
        in_specs=[pl.BlockSpec((S_, B_ // 8), lambda i: (0, i))],
        out_specs=pl.BlockSpec((S_, B_ // 8), lambda i: (0, i)),
        out_shape=jax.ShapeDtypeStruct((S_, B_), jnp.int32),
    )(xt)


def kernel(x, weight):
    B_, S_ = x.shape
    n_sb = S_ // SB
    n_jb = B_ // LB
    j_per_w = n_jb // NW
    n_pairs = S_ // 2

    x_u = _xmap(x.T)
    x_l = x_u.reshape(n_sb, SB, n_jb, LB).transpose(0, 2, 1, 3)
    table_lin = _wpack(weight.T).reshape(2 * H_PACK, D)

    mesh = plsc.VectorSubcoreMesh(core_axis_name="c", subcore_axis_name="s")

    @functools.partial(
        pl.kernel,
        mesh=mesh,
        out_type=jax.ShapeDtypeStruct((S_, D // SB, n_jb, SB, LB), jnp.float32),
        compiler_params=pltpu.CompilerParams(
            use_tc_tiling_on_sc=False, needs_layout_passes=False
        ),
        scratch_types=[
            pltpu.VMEM((n_sb, SB, LB), jnp.int32),
            pltpu.VMEM((LB, D), jnp.float32),
            pltpu.VMEM((LB, D), jnp.float32),
            pltpu.VMEM((LB, D), jnp.float32),
            pltpu.VMEM((LB, D), jnp.float32),
            pltpu.VMEM((D // SB, SB, LB + 1), jnp.float32),
            pltpu.VMEM((D // SB, SB, LB + 1), jnp.float32),
            pltpu.SemaphoreType.DMA,
            pltpu.SemaphoreType.DMA,
            pltpu.SemaphoreType.DMA,
            pltpu.SemaphoreType.DMA,
            pltpu.SemaphoreType.DMA,
            pltpu.SemaphoreType.DMA,
            pltpu.SemaphoreType.DMA,
        ],
    )
    def k(x_hbm, table_hbm, out_hbm, idx_v, rows0, rows1, rows2, rows3,
          tr0, tr1, isem, gsem0, gsem1, gsem2, gsem3, osem0, osem1):
        rows = (rows0, rows1, rows2, rows3)
        trs = (tr0, tr1)
        gsems = (gsem0, gsem1, gsem2, gsem3)
        osems = (osem0, osem1)
        wid = lax.axis_index("s") * NC + lax.axis_index("c")
        lane_iota = lax.iota(jnp.int32, 16)

        def gather(s, p):
            return pltpu.async_copy(
                table_hbm.at[idx_v.at[s // SB, s % SB]], rows[p], gsems[p]
            )

        d_coords = [
            ((16 * k + lane_iota) // SB, (16 * k + lane_iota) % SB)
            for k in range(D // 16)
        ]

        def transpose_from(p, t):
            def b_body(b0, carry):
                for half in range(2):
                    bs = [b0 * 8 + half * 4 + bb for bb in range(4)]
                    vs = []
                    for b in bs:
                        for k in range(D // 16):
                            vs.append(rows[p][b, pl.ds(16 * k, 16)])
                    i = 0
                    for b in bs:
                        b_splat = jnp.zeros((16,), jnp.int32) + b
                        for k in range(D // 16):
                            plsc.store_scatter(
                                trs[t],
                                [d_coords[k][0], d_coords[k][1], b_splat],
                                vs[i],
                            )
                            i += 1
                return carry

            lax.fori_loop(0, LB // 8, b_body, 0)

        def run_j(j):
            pltpu.sync_copy(x_hbm.at[:, j], idx_v)

            def out_at(s):
                return out_hbm.at[s, :, j]

            for p in range(4):
                gather(p, p)

            def quad_body(g, carry):
                for p in range(4):
                    s = 4 * g + p
                    t = p % 2
                    pltpu.make_async_copy(
                        table_hbm.at[idx_v.at[0, 0]], rows[p], gsems[p]
                    ).wait()
                    @pl.when(s > 1)
                    def _():
                        pltpu.make_async_copy(
                            out_at(s), trs[t].at[:, :, pl.ds(0, LB)], osems[t]
                        ).wait()
                    transpose_from(p, t)
                    @pl.when(s + 4 < S_)
                    def _():
                        gather(s + 4, p)
                    pltpu.async_copy(
                        trs[t].at[:, :, pl.ds(0, LB)], out_at(s), osems[t]
                    )
                return carry

            lax.fori_loop(0, S_ // 4, quad_body, 0)

            for p in range(2):
                pltpu.make_async_copy(
                    out_at(p), trs[p].at[:, :, pl.ds(0, LB)], osems[p]
                ).wait()

        for jl in range(j_per_w):
            run_j(wid * j_per_w + jl)

    out_l = k(x_l, table_lin)
    return out_l.transpose(2, 4, 0, 1, 3).reshape(B_, S_, D)


# speedup vs baseline: 1.3029x; 1.0122x over previous
import functools
import math

import jax
import jax.numpy as jnp
from jax import lax
from jax.experimental import pallas as pl
from jax.experimental.pallas import tpu as pltpu
from jax.experimental.pallas import tpu_sc as plsc

D = 64
NC = 2
NS = 16
NW = NC * NS
LB = 128
SB = 8


H_PACK = 507904
NUM_ROWS_STATIC = 1000000
RB = 16384


def _wpack(wt):
    G = H_PACK // RB

    def body(x1_ref, x2_ref, o_ref):
        o_ref[...] = jnp.concatenate([x1_ref[...].T, x2_ref[...].T], axis=1)

    last_blk = (NUM_ROWS_STATIC - 1) // RB

    return pl.pallas_call(
        body, grid=(G,),
        in_specs=[pl.BlockSpec((D, RB), lambda i: (0, i)),
                  pl.BlockSpec((D, RB),
                               lambda i: (0, jnp.minimum(i + G, last_blk)))],
        out_specs=pl.BlockSpec((RB, 2 * D), lambda i: (i, 0)),
        out_shape=jax.ShapeDtypeStruct((H_PACK, 2 * D), jnp.float32),
    )(wt, wt)


def _xmap(xt):
    def body(x_ref, o_ref):
        v = x_ref[...]
        ge = (v >= H_PACK).astype(jnp.int32)
        o_ref[...] = 2 * v - (2 * H_PACK - 1) * ge

    S_, B_ = xt.shape
    return pl.pallas_call(
        body, grid=(8,),
        in_specs=[pl.BlockSpec((S_, B_ // 8), lambda i: (0, i))],
        out_specs=pl.BlockSpec((S_, B_ // 8), lambda i: (0, i)),
        out_shape=jax.ShapeDtypeStruct((S_, B_), jnp.int32),
    )(xt)


def kernel(x, weight):
    B_, S_ = x.shape
    n_sb = S_ // SB
    n_jb = B_ // LB
    j_per_w = n_jb // NW
    n_pairs = S_ // 2

    x_u = _xmap(x.T)
    x_l = x_u.reshape(n_sb, SB, n_jb, LB).transpose(0, 2, 1, 3)
    table_lin = _wpack(weight.T).reshape(2 * H_PACK, D)

    mesh = plsc.VectorSubcoreMesh(core_axis_name="c", subcore_axis_name="s")

    @functools.partial(
        pl.kernel,
        mesh=mesh,
        out_type=jax.ShapeDtypeStruct((S_, D // SB, n_jb, SB, LB), jnp.float32),
        compiler_params=pltpu.CompilerParams(
            use_tc_tiling_on_sc=False, needs_layout_passes=False
        ),
        scratch_types=[
            pltpu.VMEM((n_sb, SB, LB), jnp.int32),
            pltpu.VMEM((LB, D), jnp.float32),
            pltpu.VMEM((LB, D), jnp.float32),
            pltpu.VMEM((LB, D), jnp.float32),
            pltpu.VMEM((LB, D), jnp.float32),
            pltpu.VMEM((D // SB, SB, LB + 1), jnp.float32),
            pltpu.VMEM((D // SB, SB, LB + 1), jnp.float32),
            pltpu.SemaphoreType.DMA,
            pltpu.SemaphoreType.DMA,
            pltpu.SemaphoreType.DMA,
            pltpu.SemaphoreType.DMA,
            pltpu.SemaphoreType.DMA,
            pltpu.SemaphoreType.DMA,
            pltpu.SemaphoreType.DMA,
        ],
    )
    def k(x_hbm, table_hbm, out_hbm, idx_v, rows0, rows1, rows2, rows3,
          tr0, tr1, isem, gsem0, gsem1, gsem2, gsem3, osem0, osem1):
        rows = (rows0, rows1, rows2, rows3)
        trs = (tr0, tr1)
        gsems = (gsem0, gsem1, gsem2, gsem3)
        osems = (osem0, osem1)
        wid = lax.axis_index("s") * NC + lax.axis_index("c")
        lane_iota = lax.iota(jnp.int32, 16)

        def gather(s, p):
            return pltpu.async_copy(
                table_hbm.at[idx_v.at[s // SB, s % SB]], rows[p], gsems[p]
            )

        d_coords = [
            ((16 * k + lane_iota) // SB, (16 * k + lane_iota) % SB)
            for k in range(D // 16)
        ]

        def transpose_from(p, t):
            def b_body(b0, carry):
                for half in range(2):
                    bs = [b0 * 8 + half * 4 + bb for bb in range(4)]
                    vs = []
                    for b in bs:
                        for k in range(D // 16):
                            vs.append(rows[p][b, pl.ds(16 * k, 16)])
                    i = 0
                    for b in bs:
                        b_splat = jnp.zeros((16,), jnp.int32) + b
                        for k in range(D // 16):
                            plsc.store_scatter(
                                trs[t],
                                [d_coords[k][0], d_coords[k][1], b_splat],
                                vs[i],
                            )
                            i += 1
                return carry

            lax.fori_loop(0, LB // 8, b_body, 0)

        def run_j(j):
            pltpu.sync_copy(x_hbm.at[:, j], idx_v)

            def out_at(s):
                return out_hbm.at[s, :, j]

            for p in range(4):
                gather(p, p)

            def quad_body(g, carry):
                for p in range(4):
                    s = 4 * g + p
                    t = p % 2
                    pltpu.make_async_copy(
                        table_hbm.at[idx_v.at[0, 0]], rows[p], gsems[p]
                    ).wait()
                    @pl.when(s > 1)
                    def _():
                        pltpu.make_async_copy(
                            out_at(s), trs[t].at[:, :, pl.ds(0, LB)], osems[t]
                        ).wait()
                    transpose_from(p, t)
                    @pl.when(s + 4 < S_)
                    def _():
                        gather(s + 4, p)
                    pltpu.async_copy(
                        trs[t].at[:, :, pl.ds(0, LB)], out_at(s), osems[t]
                    )
                return carry

            lax.fori_loop(0, S_ // 4, quad_body, 0)

            for p in range(2):
                pltpu.make_async_copy(
                    out_at(p), trs[p].at[:, :, pl.ds(0, LB)], osems[p]
                ).wait()

        for jl in range(j_per_w):
            run_j(wid * j_per_w + jl)

    out_l = k(x_l, table_lin)
    return out_l.transpose(2, 4, 0, 1, 3).reshape(B_, S_, D)
